# baseline (speedup 1.0000x reference)
import functools

import jax
import jax.numpy as jnp
from jax import lax
from jax.experimental import pallas as pl
from jax.experimental.pallas import tpu as pltpu
from jax.experimental.pallas import tpu_sc as plsc

N = 10000
E = 320000
CH = 128
PE_DIM = 20
PE_EMB = 28
NODE_VOCAB = 64
A = 16
L = 3

NC = 2
NS = 16
NW = NC * NS
CK = 128
NCHUNK = 80
EW = NCHUNK * CK
EPAD = NW * EW
NP = 10240
RPS = NP // NS

BM = 400


def _prologue_body(x_ref, pe_ref, nt_ref, g_ref, b_ref, pw_ref, pb_ref,
                   pehat_ref, h0_ref):
    pe = pe_ref[...]
    mean = jnp.mean(pe, axis=0, keepdims=True)
    var = jnp.mean((pe - mean) ** 2, axis=0, keepdims=True)
    pen = (pe - mean) / jnp.sqrt(var + 1e-5) * g_ref[...] + b_ref[...]
    pehat = jnp.dot(pen, pw_ref[...], preferred_element_type=jnp.float32)
    pehat = pehat + pb_ref[...]
    pehat_ref[...] = pehat
    onehot = (x_ref[...] == lax.broadcasted_iota(jnp.int32, (N, NODE_VOCAB), 1)
              ).astype(jnp.float32)
    hnode = jnp.dot(onehot, nt_ref[...], preferred_element_type=jnp.float32)
    h0_ref[...] = jnp.concatenate([hnode, pehat], axis=1)


def _prologue(x, pe, node_table, pe_gamma, pe_beta, pe_W, pe_b):
    return pl.pallas_call(
        _prologue_body,
        out_shape=(jax.ShapeDtypeStruct((N, PE_EMB), jnp.float32),
                   jax.ShapeDtypeStruct((N, CH), jnp.float32)),
    )(x, pe, node_table, pe_gamma.reshape(1, PE_DIM),
      pe_beta.reshape(1, PE_DIM), pe_W, pe_b.reshape(1, PE_EMB))


def _table_body(h_ref, ea_ref, t_ref):
    t_ref[...] = jnp.maximum(h_ref[...][:, None, :] + ea_ref[...][None, :, :],
                             0.0)


def _build_table(h, edge_table):
    return pl.pallas_call(
        _table_body,
        grid=(N // BM,),
        in_specs=[pl.BlockSpec((BM, CH), lambda i: (i, 0)),
                  pl.BlockSpec((A, CH), lambda i: (0, 0))],
        out_specs=pl.BlockSpec((BM, A, CH), lambda i: (i, 0, 0)),
        out_shape=jax.ShapeDtypeStruct((N, A, CH), jnp.float32),
    )(h, edge_table)


def _sc_aggr_body(t_hbm, pidx_hbm, zz_hbm, out_hbm,
                  pbuf_v, ubuf_v, rows_v, acc_sh,
                  semi0, semi1, semg0, semg1, semz):
    cid = lax.axis_index("c")
    sid = lax.axis_index("s")
    wid = sid * NC + cid

    def idxcpy(c, pb, sem):
        cc = jnp.minimum(c, NCHUNK - 1)
        return pltpu.async_copy(pidx_hbm.at[wid, cc], pbuf_v.at[pb], sem)

    def idxwait(pb, sem):
        pltpu.make_async_copy(pidx_hbm.at[wid, 0], pbuf_v.at[pb], sem).wait()

    def unpack(ub):
        for j in range(CK // 16):
            w = pbuf_v[ub, pl.ds(j * 16, 16)]
            ubuf_v[ub, 0, pl.ds(j * 16, 16)] = w & 0x3FFFF
            ubuf_v[ub, 1, pl.ds(j * 16, 16)] = lax.shift_right_logical(w, 18)

    def gather(ub, sem):
        return pltpu.async_copy(t_hbm.at[ubuf_v.at[ub, 0]], rows_v.at[ub],
                                sem)

    def gwait(ub, sem):
        pltpu.make_async_copy(t_hbm.at[ubuf_v.at[ub, 0]], rows_v.at[ub],
                              sem).wait()

    def scatter_add(ub):
        pltpu.sync_copy(rows_v.at[ub], acc_sh.at[ubuf_v.at[ub, 1]], add=True)

    zcp = pltpu.async_copy(zz_hbm.at[pl.ds(sid * RPS, RPS)],
                           acc_sh.at[pl.ds(sid * RPS, RPS)], semz)
    idxcpy(0, 0, semi0)
    idxwait(0, semi0)
    unpack(0)
    idxcpy(1, 1, semi1)
    idxcpy(2, 0, semi0)
    zcp.wait()
    plsc.subcore_barrier()
    gather(0, semg0)

    def body(i, carry):
        idxwait(1, semi1)
        unpack(1)
        gather(1, semg1)
        idxcpy(2 * i + 3, 1, semi1)
        gwait(0, semg0)
        scatter_add(0)
        idxwait(0, semi0)
        unpack(0)
        gather(0, semg0)
        idxcpy(2 * i + 4, 0, semi0)
        gwait(1, semg1)
        scatter_add(1)
        return carry

    lax.fori_loop(0, NCHUNK // 2 - 1, body, 0)
    idxwait(1, semi1)
    unpack(1)
    gather(1, semg1)
    idxwait(0, semi0)
    gwait(0, semg0)
    scatter_add(0)
    gwait(1, semg1)
    scatter_add(1)
    plsc.subcore_barrier()
    pltpu.sync_copy(acc_sh.at[pl.ds(sid * RPS, RPS)],
                    out_hbm.at[cid, pl.ds(sid * RPS, RPS)])


@functools.partial(
    pl.kernel,
    out_type=jax.ShapeDtypeStruct((NC, NP, CH), jnp.float32),
    mesh=plsc.VectorSubcoreMesh(core_axis_name="c", subcore_axis_name="s"),
    scratch_types=[
        pltpu.VMEM((2, CK), jnp.int32),
        pltpu.VMEM((2, 2, CK), jnp.int32),
        pltpu.VMEM((2, CK, CH), jnp.float32),
        pltpu.VMEM_SHARED((NP, CH), jnp.float32),
        pltpu.SemaphoreType.DMA,
        pltpu.SemaphoreType.DMA,
        pltpu.SemaphoreType.DMA,
        pltpu.SemaphoreType.DMA,
        pltpu.SemaphoreType.DMA,
    ],
)
def _sc_aggr(t_hbm, pidx_hbm, zz_hbm, out_hbm,
             pbuf_v, ubuf_v, rows_v, acc_sh, semi0, semi1, semg0, semg1,
             semz):
    _sc_aggr_body(t_hbm, pidx_hbm, zz_hbm, out_hbm,
                  pbuf_v, ubuf_v, rows_v, acc_sh, semi0, semi1, semg0,
                  semg1, semz)


def _mlp_table_body(h_ref, ag_ref, w1_ref, b1_ref, w2_ref, b2_ref, ea_ref,
                    ho_ref, t_ref):
    z = h_ref[...] + ag_ref[0] + ag_ref[1]
    u = jnp.maximum(jnp.dot(z, w1_ref[...], preferred_element_type=jnp.float32)
                    + b1_ref[...], 0.0)
    hn = jnp.dot(u, w2_ref[...], preferred_element_type=jnp.float32) + b2_ref[...]
    ho_ref[...] = hn
    t_ref[...] = jnp.maximum(hn[:, None, :] + ea_ref[...][None, :, :], 0.0)


def _mlp_body(h_ref, ag_ref, w1_ref, b1_ref, w2_ref, b2_ref, ho_ref):
    z = h_ref[...] + ag_ref[0] + ag_ref[1]
    u = jnp.maximum(jnp.dot(z, w1_ref[...], preferred_element_type=jnp.float32)
                    + b1_ref[...], 0.0)
    ho_ref[...] = jnp.dot(u, w2_ref[...], preferred_element_type=jnp.float32) \
        + b2_ref[...]


def _mlp(h, aggr, w1, b1, w2, b2, edge_table, build_table):
    in_specs = [pl.BlockSpec((BM, CH), lambda i: (i, 0)),
                pl.BlockSpec((NC, BM, CH), lambda i: (0, i, 0)),
                pl.BlockSpec((CH, CH), lambda i: (0, 0)),
                pl.BlockSpec((1, CH), lambda i: (0, 0)),
                pl.BlockSpec((CH, CH), lambda i: (0, 0)),
                pl.BlockSpec((1, CH), lambda i: (0, 0))]
    args = (h, aggr, w1, b1.reshape(1, CH), w2, b2.reshape(1, CH))
    if build_table:
        return pl.pallas_call(
            _mlp_table_body,
            grid=(N // BM,),
            in_specs=in_specs + [pl.BlockSpec((A, CH), lambda i: (0, 0))],
            out_specs=(pl.BlockSpec((BM, CH), lambda i: (i, 0)),
                       pl.BlockSpec((BM, A, CH), lambda i: (i, 0, 0))),
            out_shape=(jax.ShapeDtypeStruct((N, CH), jnp.float32),
                       jax.ShapeDtypeStruct((N, A, CH), jnp.float32)),
        )(*args, edge_table)
    h2 = pl.pallas_call(
        _mlp_body,
        grid=(N // BM,),
        in_specs=in_specs,
        out_specs=pl.BlockSpec((BM, CH), lambda i: (i, 0)),
        out_shape=jax.ShapeDtypeStruct((N, CH), jnp.float32),
    )(*args)
    return h2, None


def kernel(x, pe, edge_index, edge_attr, batch, node_table, edge_table,
           pe_gamma, pe_beta, pe_W, pe_b, W1, b1, W2, b2):
    pe_hat, h = _prologue(x, pe, node_table, pe_gamma, pe_beta, pe_W, pe_b)
    t = _build_table(h, edge_table)

    src = edge_index[0]
    dst = edge_index[1]
    fidx = (src * A + edge_attr).astype(jnp.uint32)
    packed = fidx | (dst.astype(jnp.uint32) << 18)
    pad = (jnp.uint32(N) + jnp.arange(EPAD - E, dtype=jnp.uint32)
           % jnp.uint32(NP - N)) << 18
    pidx = lax.bitcast_convert_type(
        jnp.concatenate([packed, pad]), jnp.int32
    ).reshape(NW, NCHUNK, CK)
    zz = jnp.zeros((NP, CH), jnp.float32)

    for l in range(L):
        aggr = _sc_aggr(t.reshape(N * A, CH), pidx, zz)
        h, t = _mlp(h, aggr, W1[l], b1[l], W2[l], b2[l], edge_table,
                    build_table=(l < L - 1))
    return (h, pe_hat)


# speedup vs baseline: 2.5611x; 2.5611x over previous
import functools

import jax
import jax.numpy as jnp
from jax import lax
from jax.experimental import pallas as pl
from jax.experimental.pallas import tpu as pltpu
from jax.experimental.pallas import tpu_sc as plsc

N = 10000
E = 320000
CH = 128
PE_DIM = 20
PE_EMB = 28
NODE_VOCAB = 64
A = 16
L = 3

NC = 2
NS = 16
NW = NC * NS
CK = 128
NCHUNK = 80
EW = NCHUNK * CK
EPAD = NW * EW
NP = 10240
RPS = NP // NS

BM = 400


def _prologue_body(x_ref, pe_ref, nt_ref, g_ref, b_ref, pw_ref, pb_ref,
                   pehat_ref, h0_ref):
    pe = pe_ref[...]
    mean = jnp.mean(pe, axis=0, keepdims=True)
    var = jnp.mean((pe - mean) ** 2, axis=0, keepdims=True)
    pen = (pe - mean) / jnp.sqrt(var + 1e-5) * g_ref[...] + b_ref[...]
    pehat = jnp.dot(pen, pw_ref[...], preferred_element_type=jnp.float32)
    pehat = pehat + pb_ref[...]
    pehat_ref[...] = pehat
    onehot = (x_ref[...] == lax.broadcasted_iota(jnp.int32, (N, NODE_VOCAB), 1)
              ).astype(jnp.float32)
    hnode = jnp.dot(onehot, nt_ref[...], preferred_element_type=jnp.float32)
    h0_ref[...] = jnp.concatenate([hnode, pehat], axis=1)


def _prologue(x, pe, node_table, pe_gamma, pe_beta, pe_W, pe_b):
    return pl.pallas_call(
        _prologue_body,
        out_shape=(jax.ShapeDtypeStruct((N, PE_EMB), jnp.float32),
                   jax.ShapeDtypeStruct((N, CH), jnp.float32)),
    )(x, pe, node_table, pe_gamma.reshape(1, PE_DIM),
      pe_beta.reshape(1, PE_DIM), pe_W, pe_b.reshape(1, PE_EMB))


def _table_body(h_ref, ea_ref, t_ref):
    t_ref[...] = jnp.maximum(h_ref[...][:, None, :] + ea_ref[...][None, :, :],
                             0.0)


def _build_table(h, edge_table):
    return pl.pallas_call(
        _table_body,
        grid=(N // BM,),
        in_specs=[pl.BlockSpec((BM, CH), lambda i: (i, 0)),
                  pl.BlockSpec((A, CH), lambda i: (0, 0))],
        out_specs=pl.BlockSpec((BM, A, CH), lambda i: (i, 0, 0)),
        out_shape=jax.ShapeDtypeStruct((N, A, CH), jnp.float32),
    )(h, edge_table)


def _sc_aggr_body(t_hbm, pidx_hbm, zz_hbm, out_hbm,
                  pbuf_v, ubuf_v, rows_v, acc_sh,
                  semi0, semi1, semg0, semg1, semz):
    cid = lax.axis_index("c")
    sid = lax.axis_index("s")
    wid = sid * NC + cid

    def idxcpy(c, pb, sem):
        cc = jnp.minimum(c, NCHUNK - 1)
        return pltpu.async_copy(pidx_hbm.at[wid, cc], pbuf_v.at[pb], sem)

    def idxwait(pb, sem):
        pltpu.make_async_copy(pidx_hbm.at[wid, 0], pbuf_v.at[pb], sem).wait()

    def unpack(ub):
        for j in range(CK // 16):
            w = pbuf_v[ub, pl.ds(j * 16, 16)]
            ubuf_v[ub, 0, pl.ds(j * 16, 16)] = w & 0x3FFFF
            ubuf_v[ub, 1, pl.ds(j * 16, 16)] = lax.shift_right_logical(w, 18)

    def gather(ub, sem):
        return pltpu.async_copy(t_hbm.at[ubuf_v.at[ub, 0]], rows_v.at[ub],
                                sem)

    def gwait(ub, sem):
        pltpu.make_async_copy(t_hbm.at[ubuf_v.at[ub, 0]], rows_v.at[ub],
                              sem).wait()

    def scatter_add(ub):
        pltpu.sync_copy(rows_v.at[ub], acc_sh.at[ubuf_v.at[ub, 1]], add=True)

    zcp = pltpu.async_copy(zz_hbm.at[pl.ds(sid * RPS, RPS)],
                           acc_sh.at[pl.ds(sid * RPS, RPS)], semz)
    idxcpy(0, 0, semi0)
    idxwait(0, semi0)
    unpack(0)
    idxcpy(1, 1, semi1)
    idxcpy(2, 0, semi0)
    zcp.wait()
    plsc.subcore_barrier()
    gather(0, semg0)

    def body(i, carry):
        idxwait(1, semi1)
        unpack(1)
        gather(1, semg1)
        idxcpy(2 * i + 3, 1, semi1)
        gwait(0, semg0)
        scatter_add(0)
        idxwait(0, semi0)
        unpack(0)
        gather(0, semg0)
        idxcpy(2 * i + 4, 0, semi0)
        gwait(1, semg1)
        scatter_add(1)
        return carry

    lax.fori_loop(0, NCHUNK // 2 - 1, body, 0)
    idxwait(1, semi1)
    unpack(1)
    gather(1, semg1)
    idxwait(0, semi0)
    gwait(0, semg0)
    scatter_add(0)
    gwait(1, semg1)
    scatter_add(1)
    plsc.subcore_barrier()
    pltpu.sync_copy(acc_sh.at[pl.ds(sid * RPS, RPS)],
                    out_hbm.at[cid, pl.ds(sid * RPS, RPS)])


@functools.partial(
    pl.kernel,
    out_type=jax.ShapeDtypeStruct((NC, NP, CH), jnp.float32),
    mesh=plsc.VectorSubcoreMesh(core_axis_name="c", subcore_axis_name="s"),
    scratch_types=[
        pltpu.VMEM((2, CK), jnp.int32),
        pltpu.VMEM((2, 2, CK), jnp.int32),
        pltpu.VMEM((2, CK, CH), jnp.float32),
        pltpu.VMEM_SHARED((NP, CH), jnp.float32),
        pltpu.SemaphoreType.DMA,
        pltpu.SemaphoreType.DMA,
        pltpu.SemaphoreType.DMA,
        pltpu.SemaphoreType.DMA,
        pltpu.SemaphoreType.DMA,
    ],
)
def _sc_aggr(t_hbm, pidx_hbm, zz_hbm, out_hbm,
             pbuf_v, ubuf_v, rows_v, acc_sh, semi0, semi1, semg0, semg1,
             semz):
    _sc_aggr_body(t_hbm, pidx_hbm, zz_hbm, out_hbm,
                  pbuf_v, ubuf_v, rows_v, acc_sh, semi0, semi1, semg0,
                  semg1, semz)


def _mlp_table_body(h_ref, ag_ref, w1_ref, b1_ref, w2_ref, b2_ref, ea_ref,
                    ho_ref, t_ref):
    z = h_ref[...] + ag_ref[0] + ag_ref[1]
    u = jnp.maximum(jnp.dot(z, w1_ref[...], preferred_element_type=jnp.float32)
                    + b1_ref[...], 0.0)
    hn = jnp.dot(u, w2_ref[...], preferred_element_type=jnp.float32) + b2_ref[...]
    ho_ref[...] = hn
    t_ref[...] = jnp.maximum(hn[:, None, :] + ea_ref[...][None, :, :], 0.0)


def _mlp_body(h_ref, ag_ref, w1_ref, b1_ref, w2_ref, b2_ref, ho_ref):
    z = h_ref[...] + ag_ref[0] + ag_ref[1]
    u = jnp.maximum(jnp.dot(z, w1_ref[...], preferred_element_type=jnp.float32)
                    + b1_ref[...], 0.0)
    ho_ref[...] = jnp.dot(u, w2_ref[...], preferred_element_type=jnp.float32) \
        + b2_ref[...]


def _mlp(h, aggr, w1, b1, w2, b2, edge_table, build_table):
    in_specs = [pl.BlockSpec((BM, CH), lambda i: (i, 0)),
                pl.BlockSpec((NC, BM, CH), lambda i: (0, i, 0)),
                pl.BlockSpec((CH, CH), lambda i: (0, 0)),
                pl.BlockSpec((1, CH), lambda i: (0, 0)),
                pl.BlockSpec((CH, CH), lambda i: (0, 0)),
                pl.BlockSpec((1, CH), lambda i: (0, 0))]
    args = (h, aggr, w1, b1.reshape(1, CH), w2, b2.reshape(1, CH))
    if build_table:
        return pl.pallas_call(
            _mlp_table_body,
            grid=(N // BM,),
            in_specs=in_specs + [pl.BlockSpec((A, CH), lambda i: (0, 0))],
            out_specs=(pl.BlockSpec((BM, CH), lambda i: (i, 0)),
                       pl.BlockSpec((BM, A, CH), lambda i: (i, 0, 0))),
            out_shape=(jax.ShapeDtypeStruct((N, CH), jnp.float32),
                       jax.ShapeDtypeStruct((N, A, CH), jnp.float32)),
        )(*args, edge_table)
    h2 = pl.pallas_call(
        _mlp_body,
        grid=(N // BM,),
        in_specs=in_specs,
        out_specs=pl.BlockSpec((BM, CH), lambda i: (i, 0)),
        out_shape=jax.ShapeDtypeStruct((N, CH), jnp.float32),
    )(*args)
    return h2, None


def kernel(x, pe, edge_index, edge_attr, batch, node_table, edge_table,
           pe_gamma, pe_beta, pe_W, pe_b, W1, b1, W2, b2):
    pe_hat, h = _prologue(x, pe, node_table, pe_gamma, pe_beta, pe_W, pe_b)
    t = _build_table(h, edge_table)

    src = edge_index[0]
    dst = edge_index[1]
    fidx = (src * A + edge_attr).astype(jnp.uint32)
    packed = fidx | (dst.astype(jnp.uint32) << 18)
    park = jnp.arange(EPAD - E, dtype=jnp.uint32)
    pad = (((jnp.uint32(N) + park % jnp.uint32(NP - N)) << 18)
           | (park * jnp.uint32(331) % jnp.uint32(N * A)))
    pidx = lax.bitcast_convert_type(
        jnp.concatenate([packed, pad]), jnp.int32
    ).reshape(NW, NCHUNK, CK)
    zz = jnp.zeros((NP, CH), jnp.float32)

    for l in range(L):
        aggr = _sc_aggr(t.reshape(N * A, CH), pidx, zz)
        h, t = _mlp(h, aggr, W1[l], b1[l], W2[l], b2[l], edge_table,
                    build_table=(l < L - 1))
    return (h, pe_hat)


# speedup vs baseline: 2.6582x; 1.0379x over previous
import functools

import jax
import jax.numpy as jnp
from jax import lax
from jax.experimental import pallas as pl
from jax.experimental.pallas import tpu as pltpu
from jax.experimental.pallas import tpu_sc as plsc

N = 10000
E = 320000
CH = 128
PE_DIM = 20
PE_EMB = 28
NODE_VOCAB = 64
A = 16
L = 3

NC = 2
NS = 16
NW = NC * NS
CK = 64
NCHUNK = 159
EW = NCHUNK * CK
EPAD = NW * EW
NP = 10240
RPS = NP // NS

BM = 400


def _prologue_body(x_ref, pe_ref, nt_ref, g_ref, b_ref, pw_ref, pb_ref,
                   pehat_ref, h0_ref):
    pe = pe_ref[...]
    mean = jnp.mean(pe, axis=0, keepdims=True)
    var = jnp.mean((pe - mean) ** 2, axis=0, keepdims=True)
    pen = (pe - mean) / jnp.sqrt(var + 1e-5) * g_ref[...] + b_ref[...]
    pehat = jnp.dot(pen, pw_ref[...], preferred_element_type=jnp.float32)
    pehat = pehat + pb_ref[...]
    pehat_ref[...] = pehat
    onehot = (x_ref[...] == lax.broadcasted_iota(jnp.int32, (N, NODE_VOCAB), 1)
              ).astype(jnp.float32)
    hnode = jnp.dot(onehot, nt_ref[...], preferred_element_type=jnp.float32)
    h0_ref[...] = jnp.concatenate([hnode, pehat], axis=1)


def _prologue(x, pe, node_table, pe_gamma, pe_beta, pe_W, pe_b):
    return pl.pallas_call(
        _prologue_body,
        out_shape=(jax.ShapeDtypeStruct((N, PE_EMB), jnp.float32),
                   jax.ShapeDtypeStruct((N, CH), jnp.float32)),
    )(x, pe, node_table, pe_gamma.reshape(1, PE_DIM),
      pe_beta.reshape(1, PE_DIM), pe_W, pe_b.reshape(1, PE_EMB))


def _table_body(h_ref, ea_ref, t_ref):
    t_ref[...] = jnp.maximum(h_ref[...][:, None, :] + ea_ref[...][None, :, :],
                             0.0)


def _build_table(h, edge_table):
    return pl.pallas_call(
        _table_body,
        grid=(N // BM,),
        in_specs=[pl.BlockSpec((BM, CH), lambda i: (i, 0)),
                  pl.BlockSpec((A, CH), lambda i: (0, 0))],
        out_specs=pl.BlockSpec((BM, A, CH), lambda i: (i, 0, 0)),
        out_shape=jax.ShapeDtypeStruct((N, A, CH), jnp.float32),
    )(h, edge_table)


def _sc_aggr_body(t_hbm, pidx_hbm, zz_hbm, out_hbm,
                  pidx_v, ubuf_v, rows_v, acc_sh,
                  semg, sems, semz):
    cid = lax.axis_index("c")
    sid = lax.axis_index("s")
    wid = sid * NC + cid

    def unpack(c, b):
        for j in range(CK // 16):
            w = pidx_v[c, pl.ds(j * 16, 16)]
            ubuf_v[b, 0, pl.ds(j * 16, 16)] = w & 0x3FFFF
            ubuf_v[b, 1, pl.ds(j * 16, 16)] = lax.shift_right_logical(w, 18)

    def gather(b):
        pltpu.async_copy(t_hbm.at[ubuf_v.at[b, 0]], rows_v.at[b], semg[b])

    def gwait(b):
        pltpu.make_async_copy(t_hbm.at[ubuf_v.at[b, 0]], rows_v.at[b],
                              semg[b]).wait()

    def sstart(b):
        pltpu.async_copy(rows_v.at[b], acc_sh.at[ubuf_v.at[b, 1]], sems[b],
                         add=True)

    def swait(b):
        pltpu.make_async_copy(rows_v.at[b], acc_sh.at[ubuf_v.at[b, 1]],
                              sems[b]).wait()

    zcp = pltpu.async_copy(zz_hbm.at[pl.ds(sid * RPS, RPS)],
                           acc_sh.at[pl.ds(sid * RPS, RPS)], semz)
    pltpu.sync_copy(pidx_hbm.at[wid], pidx_v)
    unpack(0, 0)
    unpack(1, 1)
    zcp.wait()
    plsc.subcore_barrier()

    gather(0)
    gather(1)
    unpack(2, 2)
    gather(2)
    gwait(0)
    sstart(0)

    def body(j, carry):
        for k in range(3):
            c = 3 * j + k
            swait(k)
            unpack(c, k)
            gather(k)
            gwait((k + 1) % 3)
            sstart((k + 1) % 3)
        return carry

    lax.fori_loop(1, NCHUNK // 3, body, 0)
    gwait(1)
    sstart(1)
    gwait(2)
    sstart(2)
    swait(0)
    swait(1)
    swait(2)
    plsc.subcore_barrier()
    pltpu.sync_copy(acc_sh.at[pl.ds(sid * RPS, RPS)],
                    out_hbm.at[cid, pl.ds(sid * RPS, RPS)])


@functools.partial(
    pl.kernel,
    out_type=jax.ShapeDtypeStruct((NC, NP, CH), jnp.float32),
    mesh=plsc.VectorSubcoreMesh(core_axis_name="c", subcore_axis_name="s"),
    scratch_types=[
        pltpu.VMEM((NCHUNK, CK), jnp.int32),
        pltpu.VMEM((3, 2, CK), jnp.int32),
        pltpu.VMEM((3, CK, CH), jnp.float32),
        pltpu.VMEM_SHARED((NP, CH), jnp.float32),
        pltpu.SemaphoreType.DMA,
        pltpu.SemaphoreType.DMA,
        pltpu.SemaphoreType.DMA,
        pltpu.SemaphoreType.DMA,
        pltpu.SemaphoreType.DMA,
        pltpu.SemaphoreType.DMA,
        pltpu.SemaphoreType.DMA,
    ],
)
def _sc_aggr(t_hbm, pidx_hbm, zz_hbm, out_hbm,
             pidx_v, ubuf_v, rows_v, acc_sh, sg0, sg1, sg2, ss0, ss1, ss2,
             semz):
    _sc_aggr_body(t_hbm, pidx_hbm, zz_hbm, out_hbm,
                  pidx_v, ubuf_v, rows_v, acc_sh,
                  (sg0, sg1, sg2), (ss0, ss1, ss2), semz)


def _mlp_table_body(h_ref, ag_ref, w1_ref, b1_ref, w2_ref, b2_ref, ea_ref,
                    ho_ref, t_ref):
    z = h_ref[...] + ag_ref[0] + ag_ref[1]
    u = jnp.maximum(jnp.dot(z, w1_ref[...], preferred_element_type=jnp.float32)
                    + b1_ref[...], 0.0)
    hn = jnp.dot(u, w2_ref[...], preferred_element_type=jnp.float32) + b2_ref[...]
    ho_ref[...] = hn
    t_ref[...] = jnp.maximum(hn[:, None, :] + ea_ref[...][None, :, :], 0.0)


def _mlp_body(h_ref, ag_ref, w1_ref, b1_ref, w2_ref, b2_ref, ho_ref):
    z = h_ref[...] + ag_ref[0] + ag_ref[1]
    u = jnp.maximum(jnp.dot(z, w1_ref[...], preferred_element_type=jnp.float32)
                    + b1_ref[...], 0.0)
    ho_ref[...] = jnp.dot(u, w2_ref[...], preferred_element_type=jnp.float32) \
        + b2_ref[...]


def _mlp(h, aggr, w1, b1, w2, b2, edge_table, build_table):
    in_specs = [pl.BlockSpec((BM, CH), lambda i: (i, 0)),
                pl.BlockSpec((NC, BM, CH), lambda i: (0, i, 0)),
                pl.BlockSpec((CH, CH), lambda i: (0, 0)),
                pl.BlockSpec((1, CH), lambda i: (0, 0)),
                pl.BlockSpec((CH, CH), lambda i: (0, 0)),
                pl.BlockSpec((1, CH), lambda i: (0, 0))]
    args = (h, aggr, w1, b1.reshape(1, CH), w2, b2.reshape(1, CH))
    if build_table:
        return pl.pallas_call(
            _mlp_table_body,
            grid=(N // BM,),
            in_specs=in_specs + [pl.BlockSpec((A, CH), lambda i: (0, 0))],
            out_specs=(pl.BlockSpec((BM, CH), lambda i: (i, 0)),
                       pl.BlockSpec((BM, A, CH), lambda i: (i, 0, 0))),
            out_shape=(jax.ShapeDtypeStruct((N, CH), jnp.float32),
                       jax.ShapeDtypeStruct((N, A, CH), jnp.float32)),
        )(*args, edge_table)
    h2 = pl.pallas_call(
        _mlp_body,
        grid=(N // BM,),
        in_specs=in_specs,
        out_specs=pl.BlockSpec((BM, CH), lambda i: (i, 0)),
        out_shape=jax.ShapeDtypeStruct((N, CH), jnp.float32),
    )(*args)
    return h2, None


def kernel(x, pe, edge_index, edge_attr, batch, node_table, edge_table,
           pe_gamma, pe_beta, pe_W, pe_b, W1, b1, W2, b2):
    pe_hat, h = _prologue(x, pe, node_table, pe_gamma, pe_beta, pe_W, pe_b)
    t = _build_table(h, edge_table)

    src = edge_index[0]
    dst = edge_index[1]
    fidx = (src * A + edge_attr).astype(jnp.uint32)
    packed = fidx | (dst.astype(jnp.uint32) << 18)
    park = jnp.arange(EPAD - E, dtype=jnp.uint32)
    pad = (((jnp.uint32(N) + park % jnp.uint32(NP - N)) << 18)
           | (park * jnp.uint32(331) % jnp.uint32(N * A)))
    pidx = lax.bitcast_convert_type(
        jnp.concatenate([packed, pad]), jnp.int32
    ).reshape(NW, NCHUNK, CK)
    zz = jnp.zeros((NP, CH), jnp.float32)

    for l in range(L):
        aggr = _sc_aggr(t.reshape(N * A, CH), pidx, zz)
        h, t = _mlp(h, aggr, W1[l], b1[l], W2[l], b2[l], edge_table,
                    build_table=(l < L - 1))
    return (h, pe_hat)


# speedup vs baseline: 2.8018x; 1.0540x over previous
import functools

import jax
import jax.numpy as jnp
from jax import lax
from jax.experimental import pallas as pl
from jax.experimental.pallas import tpu as pltpu
from jax.experimental.pallas import tpu_sc as plsc

N = 10000
E = 320000
CH = 128
PE_DIM = 20
PE_EMB = 28
NODE_VOCAB = 64
A = 16
L = 3

NC = 2
NS = 16
NW = NC * NS
CK = 64
NCHUNK = 159
EW = NCHUNK * CK
EPAD = NW * EW
NP = 10240
RPS = NP // NS

BM = 1000


def _prologue_body(x_ref, pe_ref, nt_ref, g_ref, b_ref, pw_ref, pb_ref,
                   pehat_ref, h0_ref):
    pe = pe_ref[...]
    mean = jnp.mean(pe, axis=0, keepdims=True)
    var = jnp.mean((pe - mean) ** 2, axis=0, keepdims=True)
    pen = (pe - mean) / jnp.sqrt(var + 1e-5) * g_ref[...] + b_ref[...]
    pehat = jnp.dot(pen, pw_ref[...], preferred_element_type=jnp.float32)
    pehat = pehat + pb_ref[...]
    pehat_ref[...] = pehat
    onehot = (x_ref[...] == lax.broadcasted_iota(jnp.int32, (N, NODE_VOCAB), 1)
              ).astype(jnp.float32)
    hnode = jnp.dot(onehot, nt_ref[...], preferred_element_type=jnp.float32)
    h0_ref[...] = jnp.concatenate([hnode, pehat], axis=1)


def _prologue(x, pe, node_table, pe_gamma, pe_beta, pe_W, pe_b):
    return pl.pallas_call(
        _prologue_body,
        out_shape=(jax.ShapeDtypeStruct((N, PE_EMB), jnp.float32),
                   jax.ShapeDtypeStruct((N, CH), jnp.float32)),
    )(x, pe, node_table, pe_gamma.reshape(1, PE_DIM),
      pe_beta.reshape(1, PE_DIM), pe_W, pe_b.reshape(1, PE_EMB))


def _table_body(h_ref, ea_ref, t_ref):
    t_ref[...] = jnp.maximum(h_ref[...][:, None, :] + ea_ref[...][None, :, :],
                             0.0)


def _build_table(h, edge_table):
    return pl.pallas_call(
        _table_body,
        grid=(N // BM,),
        in_specs=[pl.BlockSpec((BM, CH), lambda i: (i, 0)),
                  pl.BlockSpec((A, CH), lambda i: (0, 0))],
        out_specs=pl.BlockSpec((BM, A, CH), lambda i: (i, 0, 0)),
        out_shape=jax.ShapeDtypeStruct((N, A, CH), jnp.float32),
    )(h, edge_table)


def _sc_aggr_body(t_hbm, pidx_hbm, zz_hbm, out_hbm,
                  pidx_v, ubuf_v, rows_v, acc_sh,
                  semg, sems, semz):
    cid = lax.axis_index("c")
    sid = lax.axis_index("s")
    wid = sid * NC + cid

    def unpack(c, b):
        for j in range(CK // 16):
            w = pidx_v[c, pl.ds(j * 16, 16)]
            ubuf_v[b, 0, pl.ds(j * 16, 16)] = w & 0x3FFFF
            ubuf_v[b, 1, pl.ds(j * 16, 16)] = lax.shift_right_logical(w, 18)

    def gather(b):
        pltpu.async_copy(t_hbm.at[ubuf_v.at[b, 0]], rows_v.at[b], semg[b])

    def gwait(b):
        pltpu.make_async_copy(t_hbm.at[ubuf_v.at[b, 0]], rows_v.at[b],
                              semg[b]).wait()

    def sstart(b):
        pltpu.async_copy(rows_v.at[b], acc_sh.at[ubuf_v.at[b, 1]], sems[b],
                         add=True)

    def swait(b):
        pltpu.make_async_copy(rows_v.at[b], acc_sh.at[ubuf_v.at[b, 1]],
                              sems[b]).wait()

    zcp = pltpu.async_copy(zz_hbm.at[pl.ds(sid * RPS, RPS)],
                           acc_sh.at[pl.ds(sid * RPS, RPS)], semz)
    pltpu.sync_copy(pidx_hbm.at[wid], pidx_v)
    unpack(0, 0)
    unpack(1, 1)
    zcp.wait()
    plsc.subcore_barrier()

    gather(0)
    gather(1)
    unpack(2, 2)
    gather(2)
    gwait(0)
    sstart(0)

    def body(j, carry):
        for k in range(3):
            c = 3 * j + k
            swait(k)
            unpack(c, k)
            gather(k)
            gwait((k + 1) % 3)
            sstart((k + 1) % 3)
        return carry

    lax.fori_loop(1, NCHUNK // 3, body, 0)
    gwait(1)
    sstart(1)
    gwait(2)
    sstart(2)
    swait(0)
    swait(1)
    swait(2)
    plsc.subcore_barrier()
    pltpu.sync_copy(acc_sh.at[pl.ds(sid * RPS, RPS)],
                    out_hbm.at[cid, pl.ds(sid * RPS, RPS)])


@functools.partial(
    pl.kernel,
    out_type=jax.ShapeDtypeStruct((NC, NP, CH), jnp.float32),
    mesh=plsc.VectorSubcoreMesh(core_axis_name="c", subcore_axis_name="s"),
    scratch_types=[
        pltpu.VMEM((NCHUNK, CK), jnp.int32),
        pltpu.VMEM((3, 2, CK), jnp.int32),
        pltpu.VMEM((3, CK, CH), jnp.float32),
        pltpu.VMEM_SHARED((NP, CH), jnp.float32),
        pltpu.SemaphoreType.DMA,
        pltpu.SemaphoreType.DMA,
        pltpu.SemaphoreType.DMA,
        pltpu.SemaphoreType.DMA,
        pltpu.SemaphoreType.DMA,
        pltpu.SemaphoreType.DMA,
        pltpu.SemaphoreType.DMA,
    ],
)
def _sc_aggr(t_hbm, pidx_hbm, zz_hbm, out_hbm,
             pidx_v, ubuf_v, rows_v, acc_sh, sg0, sg1, sg2, ss0, ss1, ss2,
             semz):
    _sc_aggr_body(t_hbm, pidx_hbm, zz_hbm, out_hbm,
                  pidx_v, ubuf_v, rows_v, acc_sh,
                  (sg0, sg1, sg2), (ss0, ss1, ss2), semz)


def _mlp_table_body(h_ref, ag_ref, w1_ref, b1_ref, w2_ref, b2_ref, ea_ref,
                    ho_ref, t_ref):
    z = h_ref[...] + ag_ref[0] + ag_ref[1]
    u = jnp.maximum(jnp.dot(z, w1_ref[...], preferred_element_type=jnp.float32)
                    + b1_ref[...], 0.0)
    hn = jnp.dot(u, w2_ref[...], preferred_element_type=jnp.float32) + b2_ref[...]
    ho_ref[...] = hn
    t_ref[...] = jnp.maximum(hn[:, None, :] + ea_ref[...][None, :, :], 0.0)


def _mlp_body(h_ref, ag_ref, w1_ref, b1_ref, w2_ref, b2_ref, ho_ref):
    z = h_ref[...] + ag_ref[0] + ag_ref[1]
    u = jnp.maximum(jnp.dot(z, w1_ref[...], preferred_element_type=jnp.float32)
                    + b1_ref[...], 0.0)
    ho_ref[...] = jnp.dot(u, w2_ref[...], preferred_element_type=jnp.float32) \
        + b2_ref[...]


def _mlp(h, aggr, w1, b1, w2, b2, edge_table, build_table):
    in_specs = [pl.BlockSpec((BM, CH), lambda i: (i, 0)),
                pl.BlockSpec((NC, BM, CH), lambda i: (0, i, 0)),
                pl.BlockSpec((CH, CH), lambda i: (0, 0)),
                pl.BlockSpec((1, CH), lambda i: (0, 0)),
                pl.BlockSpec((CH, CH), lambda i: (0, 0)),
                pl.BlockSpec((1, CH), lambda i: (0, 0))]
    args = (h, aggr, w1, b1.reshape(1, CH), w2, b2.reshape(1, CH))
    if build_table:
        return pl.pallas_call(
            _mlp_table_body,
            grid=(N // BM,),
            in_specs=in_specs + [pl.BlockSpec((A, CH), lambda i: (0, 0))],
            out_specs=(pl.BlockSpec((BM, CH), lambda i: (i, 0)),
                       pl.BlockSpec((BM, A, CH), lambda i: (i, 0, 0))),
            out_shape=(jax.ShapeDtypeStruct((N, CH), jnp.float32),
                       jax.ShapeDtypeStruct((N, A, CH), jnp.float32)),
        )(*args, edge_table)
    h2 = pl.pallas_call(
        _mlp_body,
        grid=(N // BM,),
        in_specs=in_specs,
        out_specs=pl.BlockSpec((BM, CH), lambda i: (i, 0)),
        out_shape=jax.ShapeDtypeStruct((N, CH), jnp.float32),
    )(*args)
    return h2, None


def kernel(x, pe, edge_index, edge_attr, batch, node_table, edge_table,
           pe_gamma, pe_beta, pe_W, pe_b, W1, b1, W2, b2):
    pe_hat, h = _prologue(x, pe, node_table, pe_gamma, pe_beta, pe_W, pe_b)
    t = _build_table(h, edge_table)

    src = edge_index[0]
    dst = edge_index[1]
    fidx = (src * A + edge_attr).astype(jnp.uint32)
    packed = fidx | (dst.astype(jnp.uint32) << 18)
    park = jnp.arange(EPAD - E, dtype=jnp.uint32)
    pad = (((jnp.uint32(N) + park % jnp.uint32(NP - N)) << 18)
           | (park * jnp.uint32(331) % jnp.uint32(N * A)))
    pidx = lax.bitcast_convert_type(
        jnp.concatenate([packed, pad]), jnp.int32
    ).reshape(NW, NCHUNK, CK)
    zz = jnp.zeros((NP, CH), jnp.float32)

    for l in range(L):
        aggr = _sc_aggr(t.reshape(N * A, CH), pidx, zz)
        h, t = _mlp(h, aggr, W1[l], b1[l], W2[l], b2[l], edge_table,
                    build_table=(l < L - 1))
    return (h, pe_hat)


# speedup vs baseline: 2.8127x; 1.0039x over previous
import functools

import jax
import jax.numpy as jnp
from jax import lax
from jax.experimental import pallas as pl
from jax.experimental.pallas import tpu as pltpu
from jax.experimental.pallas import tpu_sc as plsc

N = 10000
E = 320000
CH = 128
PE_DIM = 20
PE_EMB = 28
NODE_VOCAB = 64
A = 16
L = 3

NC = 2
NS = 16
NW = NC * NS
CK = 64
NCHUNK = 159
EW = NCHUNK * CK
EPAD = NW * EW
NP = 10240
RPS = NP // NS

BM = 2000


def _prologue_body(x_ref, pe_ref, nt_ref, g_ref, b_ref, pw_ref, pb_ref,
                   pehat_ref, h0_ref):
    pe = pe_ref[...]
    mean = jnp.mean(pe, axis=0, keepdims=True)
    var = jnp.mean((pe - mean) ** 2, axis=0, keepdims=True)
    pen = (pe - mean) / jnp.sqrt(var + 1e-5) * g_ref[...] + b_ref[...]
    pehat = jnp.dot(pen, pw_ref[...], preferred_element_type=jnp.float32)
    pehat = pehat + pb_ref[...]
    pehat_ref[...] = pehat
    onehot = (x_ref[...] == lax.broadcasted_iota(jnp.int32, (N, NODE_VOCAB), 1)
              ).astype(jnp.float32)
    hnode = jnp.dot(onehot, nt_ref[...], preferred_element_type=jnp.float32)
    h0_ref[...] = jnp.concatenate([hnode, pehat], axis=1)


def _prologue(x, pe, node_table, pe_gamma, pe_beta, pe_W, pe_b):
    return pl.pallas_call(
        _prologue_body,
        out_shape=(jax.ShapeDtypeStruct((N, PE_EMB), jnp.float32),
                   jax.ShapeDtypeStruct((N, CH), jnp.float32)),
    )(x, pe, node_table, pe_gamma.reshape(1, PE_DIM),
      pe_beta.reshape(1, PE_DIM), pe_W, pe_b.reshape(1, PE_EMB))


def _table_body(h_ref, ea_ref, t_ref):
    t_ref[...] = jnp.maximum(h_ref[...][:, None, :] + ea_ref[...][None, :, :],
                             0.0)


def _build_table(h, edge_table):
    return pl.pallas_call(
        _table_body,
        grid=(N // BM,),
        in_specs=[pl.BlockSpec((BM, CH), lambda i: (i, 0)),
                  pl.BlockSpec((A, CH), lambda i: (0, 0))],
        out_specs=pl.BlockSpec((BM, A, CH), lambda i: (i, 0, 0)),
        out_shape=jax.ShapeDtypeStruct((N, A, CH), jnp.float32),
    )(h, edge_table)


def _sc_aggr_body(t_hbm, pidx_hbm, zz_hbm, out_hbm,
                  pidx_v, ubuf_v, rows_v, acc_sh,
                  semg, sems, semz):
    cid = lax.axis_index("c")
    sid = lax.axis_index("s")
    wid = sid * NC + cid

    def unpack(c, b):
        for j in range(CK // 16):
            w = pidx_v[c, pl.ds(j * 16, 16)]
            ubuf_v[b, 0, pl.ds(j * 16, 16)] = w & 0x3FFFF
            ubuf_v[b, 1, pl.ds(j * 16, 16)] = lax.shift_right_logical(w, 18)

    def gather(b):
        pltpu.async_copy(t_hbm.at[ubuf_v.at[b, 0]], rows_v.at[b], semg[b])

    def gwait(b):
        pltpu.make_async_copy(t_hbm.at[ubuf_v.at[b, 0]], rows_v.at[b],
                              semg[b]).wait()

    def sstart(b):
        pltpu.async_copy(rows_v.at[b], acc_sh.at[ubuf_v.at[b, 1]], sems[b],
                         add=True)

    def swait(b):
        pltpu.make_async_copy(rows_v.at[b], acc_sh.at[ubuf_v.at[b, 1]],
                              sems[b]).wait()

    zcp = pltpu.async_copy(zz_hbm.at[pl.ds(sid * RPS, RPS)],
                           acc_sh.at[pl.ds(sid * RPS, RPS)], semz)
    pltpu.sync_copy(pidx_hbm.at[wid], pidx_v)
    unpack(0, 0)
    unpack(1, 1)
    zcp.wait()
    plsc.subcore_barrier()

    gather(0)
    gather(1)
    unpack(2, 2)
    gather(2)
    gwait(0)
    sstart(0)

    def body(j, carry):
        for k in range(3):
            c = 3 * j + k
            swait(k)
            unpack(c, k)
            gather(k)
            gwait((k + 1) % 3)
            sstart((k + 1) % 3)
        return carry

    lax.fori_loop(1, NCHUNK // 3, body, 0)
    gwait(1)
    sstart(1)
    gwait(2)
    sstart(2)
    swait(0)
    swait(1)
    swait(2)
    plsc.subcore_barrier()
    pltpu.sync_copy(acc_sh.at[pl.ds(sid * RPS, RPS)],
                    out_hbm.at[cid, pl.ds(sid * RPS, RPS)])


@functools.partial(
    pl.kernel,
    out_type=jax.ShapeDtypeStruct((NC, NP, CH), jnp.float32),
    mesh=plsc.VectorSubcoreMesh(core_axis_name="c", subcore_axis_name="s"),
    scratch_types=[
        pltpu.VMEM((NCHUNK, CK), jnp.int32),
        pltpu.VMEM((3, 2, CK), jnp.int32),
        pltpu.VMEM((3, CK, CH), jnp.float32),
        pltpu.VMEM_SHARED((NP, CH), jnp.float32),
        pltpu.SemaphoreType.DMA,
        pltpu.SemaphoreType.DMA,
        pltpu.SemaphoreType.DMA,
        pltpu.SemaphoreType.DMA,
        pltpu.SemaphoreType.DMA,
        pltpu.SemaphoreType.DMA,
        pltpu.SemaphoreType.DMA,
    ],
)
def _sc_aggr(t_hbm, pidx_hbm, zz_hbm, out_hbm,
             pidx_v, ubuf_v, rows_v, acc_sh, sg0, sg1, sg2, ss0, ss1, ss2,
             semz):
    _sc_aggr_body(t_hbm, pidx_hbm, zz_hbm, out_hbm,
                  pidx_v, ubuf_v, rows_v, acc_sh,
                  (sg0, sg1, sg2), (ss0, ss1, ss2), semz)


def _mlp_table_body(h_ref, ag_ref, w1_ref, b1_ref, w2_ref, b2_ref, ea_ref,
                    ho_ref, t_ref):
    z = h_ref[...] + ag_ref[0] + ag_ref[1]
    u = jnp.maximum(jnp.dot(z, w1_ref[...], preferred_element_type=jnp.float32)
                    + b1_ref[...], 0.0)
    hn = jnp.dot(u, w2_ref[...], preferred_element_type=jnp.float32) + b2_ref[...]
    ho_ref[...] = hn
    t_ref[...] = jnp.maximum(hn[:, None, :] + ea_ref[...][None, :, :], 0.0)


def _mlp_body(h_ref, ag_ref, w1_ref, b1_ref, w2_ref, b2_ref, ho_ref):
    z = h_ref[...] + ag_ref[0] + ag_ref[1]
    u = jnp.maximum(jnp.dot(z, w1_ref[...], preferred_element_type=jnp.float32)
                    + b1_ref[...], 0.0)
    ho_ref[...] = jnp.dot(u, w2_ref[...], preferred_element_type=jnp.float32) \
        + b2_ref[...]


def _mlp(h, aggr, w1, b1, w2, b2, edge_table, build_table):
    in_specs = [pl.BlockSpec((BM, CH), lambda i: (i, 0)),
                pl.BlockSpec((NC, BM, CH), lambda i: (0, i, 0)),
                pl.BlockSpec((CH, CH), lambda i: (0, 0)),
                pl.BlockSpec((1, CH), lambda i: (0, 0)),
                pl.BlockSpec((CH, CH), lambda i: (0, 0)),
                pl.BlockSpec((1, CH), lambda i: (0, 0))]
    args = (h, aggr, w1, b1.reshape(1, CH), w2, b2.reshape(1, CH))
    if build_table:
        return pl.pallas_call(
            _mlp_table_body,
            grid=(N // BM,),
            in_specs=in_specs + [pl.BlockSpec((A, CH), lambda i: (0, 0))],
            out_specs=(pl.BlockSpec((BM, CH), lambda i: (i, 0)),
                       pl.BlockSpec((BM, A, CH), lambda i: (i, 0, 0))),
            out_shape=(jax.ShapeDtypeStruct((N, CH), jnp.float32),
                       jax.ShapeDtypeStruct((N, A, CH), jnp.float32)),
        )(*args, edge_table)
    h2 = pl.pallas_call(
        _mlp_body,
        grid=(N // BM,),
        in_specs=in_specs,
        out_specs=pl.BlockSpec((BM, CH), lambda i: (i, 0)),
        out_shape=jax.ShapeDtypeStruct((N, CH), jnp.float32),
    )(*args)
    return h2, None


def kernel(x, pe, edge_index, edge_attr, batch, node_table, edge_table,
           pe_gamma, pe_beta, pe_W, pe_b, W1, b1, W2, b2):
    pe_hat, h = _prologue(x, pe, node_table, pe_gamma, pe_beta, pe_W, pe_b)
    t = _build_table(h, edge_table)

    src = edge_index[0]
    dst = edge_index[1]
    fidx = (src * A + edge_attr).astype(jnp.uint32)
    packed = fidx | (dst.astype(jnp.uint32) << 18)
    park = jnp.arange(EPAD - E, dtype=jnp.uint32)
    pad = (((jnp.uint32(N) + park % jnp.uint32(NP - N)) << 18)
           | (park * jnp.uint32(331) % jnp.uint32(N * A)))
    pidx = lax.bitcast_convert_type(
        jnp.concatenate([packed, pad]), jnp.int32
    ).reshape(NW, NCHUNK, CK)
    zz = jnp.zeros((NP, CH), jnp.float32)

    for l in range(L):
        aggr = _sc_aggr(t.reshape(N * A, CH), pidx, zz)
        h, t = _mlp(h, aggr, W1[l], b1[l], W2[l], b2[l], edge_table,
                    build_table=(l < L - 1))
    return (h, pe_hat)
